# baseline (speedup 1.0000x reference)
import functools

import jax
import jax.numpy as jnp
from jax import lax
from jax.experimental import pallas as pl
from jax.experimental.pallas import tpu as pltpu
from jax.experimental.pallas import tpu_sc as plsc

NP = 51200
RPT = NP // 16
NT = 16
NC = 2
CH = 128
K = 8
TW = 16


def _mesh():
    return plsc.VectorSubcoreMesh(core_axis_name="c", subcore_axis_name="s")


def _deg_kernel_body(nch):
    def body(dst_hbm, zeros_hbm, deg_out, slab, ones, acc, sem):
        c = lax.axis_index("c")
        s = lax.axis_index("s")
        wid = c * NT + s
        pltpu.sync_copy(dst_hbm.at[wid], slab)

        def fill(i, carry):
            ones[pl.ds(i * 16, 16)] = jnp.full((16,), 1.0, jnp.float32)
            return carry
        lax.fori_loop(0, CH // 16, fill, 0)

        pltpu.sync_copy(zeros_hbm.at[pl.ds(s * RPT, RPT)],
                        acc.at[pl.ds(s * RPT, RPT)])
        plsc.subcore_barrier()

        def group(gi, carry):
            descs = []
            for b in range(K):
                d = pltpu.async_copy(ones, acc.at[slab.at[gi * K + b]],
                                     sem, add=True)
                descs.append(d)
            for d in descs:
                d.wait()
            return carry
        lax.fori_loop(0, nch // K, group, 0)

        plsc.subcore_barrier()
        pltpu.sync_copy(acc.at[pl.ds(s * RPT, RPT)],
                        deg_out.at[pl.ds(c * NP + s * RPT, RPT)])
    return body


def _edge_kernel_body(nch):
    ngroups = nch // K
    assert ngroups % 2 == 0

    def body(idx_hbm, table_hbm, zeros_hbm, s_out, ibuf, rows, acc,
             isem, gsem, ssem):
        c = lax.axis_index("c")
        s = lax.axis_index("s")
        wid = c * NT + s
        pltpu.sync_copy(zeros_hbm.at[pl.ds(s * RPT, RPT)],
                        acc.at[pl.ds(s * RPT, RPT)])
        plsc.subcore_barrier()

        pltpu.async_copy(idx_hbm.at[wid, pl.ds(0, 2 * K)], ibuf.at[0], isem)

        def outer(g2, carry):
            for slot in range(2):
                g = g2 * 2 + slot
                pltpu.make_async_copy(idx_hbm.at[wid, pl.ds(g * 2 * K, 2 * K)],
                                      ibuf.at[slot], isem).wait()

                @pl.when(g + 1 < ngroups)
                def _():
                    pltpu.async_copy(
                        idx_hbm.at[wid, pl.ds((g + 1) * 2 * K, 2 * K)],
                        ibuf.at[1 - slot], isem)

                gdescs = []
                for k in range(K):
                    d = pltpu.async_copy(table_hbm.at[ibuf.at[slot, 2 * k]],
                                         rows.at[k], gsem)
                    gdescs.append(d)
                sdescs = []
                for k in range(K):
                    gdescs[k].wait()
                    d = pltpu.async_copy(rows.at[k],
                                         acc.at[ibuf.at[slot, 2 * k + 1]],
                                         ssem, add=True)
                    sdescs.append(d)
                for d in sdescs:
                    d.wait()
            return carry
        lax.fori_loop(0, ngroups // 2, outer, 0)

        plsc.subcore_barrier()
        pltpu.sync_copy(acc.at[pl.ds(s * RPT, RPT)],
                        s_out.at[pl.ds(c * NP + s * RPT, RPT)])
    return body


def _scale_tc_kernel(xu_ref, xi_ref, w_ref, deg_ref, g_ref):
    for c, xref in enumerate((xu_ref, xi_ref)):
        h = lax.dot_general(xref[...], w_ref[c], (((0,), (0,)), ((), ())),
                            preferred_element_type=jnp.float32)
        dinv = lax.rsqrt(deg_ref[c] + 1.0)
        br = h.shape[0]
        y = jnp.concatenate(
            [h * dinv[:, None], dinv[:, None],
             jnp.zeros((br, TW - 9), jnp.float32)],
            axis=1).reshape(br // 8, 8, TW)
        g_ref[c] = jnp.concatenate([y[:, q, :] for q in range(8)], axis=1)


def _decode_tc_kernel(g_ref, s_ref, b_ref, o_ref):
    lane = lax.broadcasted_iota(jnp.int32, (128, 128), 0)
    col = lax.broadcasted_iota(jnp.int32, (128, 128), 1)
    b8 = ((lane // TW == col // TW) & (lane % TW == 8)
          & (col % TW < 8)).astype(jnp.float32)
    grp = lax.broadcasted_iota(jnp.int32, (128, 8), 0)
    gcol = lax.broadcasted_iota(jnp.int32, (128, 8), 1)
    ssum = (grp // TW == gcol).astype(jnp.float32)

    def branch(c):
        p = s_ref[c] + g_ref[c]
        dinv_b = lax.dot(g_ref[c], b8, precision=lax.Precision.HIGHEST)
        bb = jnp.concatenate([b_ref[c], jnp.zeros((8,), jnp.float32)])
        bt = jnp.concatenate([bb] * 8)
        return jnp.tanh(dinv_b * p + bt[None, :])
    prod = branch(0) * branch(1)
    rs = lax.dot(prod, ssum,
                 precision=lax.Precision.HIGHEST)
    z = rs.reshape(rs.shape[0] // TW, TW, 8)
    o_ref[...] = jnp.concatenate([z[:, a, :] for a in range(TW)], axis=1)


def kernel(x_user, adj_user, x_item, adj_item, W_user, b_user, W_item, b_item):
    n, d_in = x_user.shape
    e = adj_user.shape[1]
    nch = -(-e // (NT * CH))
    nch = -(-nch // (2 * K)) * (2 * K)
    ept = nch * CH
    pad = NT * ept - e

    def prep(adj, offset):
        src = adj[0].astype(jnp.int32) + offset
        dst = adj[1].astype(jnp.int32)
        src = jnp.concatenate(
            [src, jnp.full((pad,), offset + n, jnp.int32)])
        dst = jnp.concatenate([dst, jnp.full((pad,), n, jnp.int32)])
        return src.reshape(NT, nch, CH), dst.reshape(NT, nch, CH)

    su, du = prep(adj_user, 0)
    si, di = prep(adj_item, NP)
    src_all = jnp.concatenate([su, si], axis=0)
    dst_all = jnp.concatenate([du, di], axis=0)
    idx_all = jnp.stack([src_all, dst_all], axis=2).reshape(
        NC * NT, 2 * nch, CH)

    zeros1 = jnp.zeros((NP,), jnp.float32)
    zeros2 = jnp.zeros((NP, TW), jnp.float32)

    deg_kernel = pl.kernel(
        _deg_kernel_body(nch),
        out_type=jax.ShapeDtypeStruct((NC * NP,), jnp.float32),
        mesh=_mesh(),
        scratch_types=[
            pltpu.VMEM((nch, CH), jnp.int32),
            pltpu.VMEM((CH,), jnp.float32),
            pltpu.VMEM_SHARED((NP,), jnp.float32),
            pltpu.SemaphoreType.DMA,
        ],
        compiler_params=pltpu.CompilerParams(use_tc_tiling_on_sc=False),
    )
    deg = deg_kernel(dst_all, zeros1)

    colpad = jnp.zeros((d_in, NP - n), jnp.float32)
    xu_t = jnp.concatenate([x_user.T, colpad], axis=1)
    xi_t = jnp.concatenate([x_item.T, colpad], axis=1)
    w_all = jnp.stack([W_user, W_item])
    deg2 = deg.reshape(NC, NP)

    BR = 2048
    nb = NP // BR
    g_all = pl.pallas_call(
        _scale_tc_kernel,
        grid=(nb,),
        in_specs=[
            pl.BlockSpec((d_in, BR), lambda i: (0, i)),
            pl.BlockSpec((d_in, BR), lambda i: (0, i)),
            pl.BlockSpec((NC, d_in, 8), lambda i: (0, 0, 0)),
            pl.BlockSpec((NC, BR), lambda i: (0, i)),
        ],
        out_specs=pl.BlockSpec((NC, BR * TW // 128, 128),
                               lambda i: (0, i, 0)),
        out_shape=jax.ShapeDtypeStruct((NC, NP * TW // 128, 128),
                                       jnp.float32),
    )(xu_t, xi_t, w_all, deg2)

    table = g_all.reshape(NC * NP, TW)

    edge_kernel = pl.kernel(
        _edge_kernel_body(nch),
        out_type=jax.ShapeDtypeStruct((NC * NP, TW), jnp.float32),
        mesh=_mesh(),
        scratch_types=[
            pltpu.VMEM((2, 2 * K, CH), jnp.int32),
            pltpu.VMEM((K, CH, TW), jnp.float32),
            pltpu.VMEM_SHARED((NP, TW), jnp.float32),
            pltpu.SemaphoreType.DMA,
            pltpu.SemaphoreType.DMA,
            pltpu.SemaphoreType.DMA,
        ],
        compiler_params=pltpu.CompilerParams(use_tc_tiling_on_sc=False),
    )
    s_acc = edge_kernel(idx_all, table, zeros2)

    b_all = jnp.stack([b_user, b_item])
    score = pl.pallas_call(
        _decode_tc_kernel,
        grid=(nb,),
        in_specs=[
            pl.BlockSpec((NC, BR * TW // 128, 128), lambda i: (0, i, 0)),
            pl.BlockSpec((NC, BR * TW // 128, 128), lambda i: (0, i, 0)),
            pl.BlockSpec((NC, 8), lambda i: (0, 0)),
        ],
        out_specs=pl.BlockSpec((BR // 128, 128), lambda i: (i, 0)),
        out_shape=jax.ShapeDtypeStruct((NP // 128, 128), jnp.float32),
    )(g_all, s_acc.reshape(NC, NP * TW // 128, 128), b_all)

    return score.reshape(NP)[:n]


# speedup vs baseline: 1.0831x; 1.0831x over previous
import jax
import jax.numpy as jnp
from jax import lax
from jax.experimental import pallas as pl
from jax.experimental.pallas import tpu as pltpu
from jax.experimental.pallas import tpu_sc as plsc

NP = 51200
RPT = NP // 16
NT = 16
NC = 2
CH = 128
K = 8
TW = 8
PN = 128 // TW


def _mesh():
    return plsc.VectorSubcoreMesh(core_axis_name="c", subcore_axis_name="s")


def _deg_kernel_body(nch):
    def body(dst_hbm, zeros_hbm, deg_out, slab, ones, acc, sem):
        c = lax.axis_index("c")
        s = lax.axis_index("s")
        wid = c * NT + s
        pltpu.sync_copy(dst_hbm.at[wid], slab)

        def fill(i, carry):
            ones[pl.ds(i * 16, 16)] = jnp.full((16,), 1.0, jnp.float32)
            return carry
        lax.fori_loop(0, CH // 16, fill, 0)

        pltpu.sync_copy(zeros_hbm.at[pl.ds(s * RPT, RPT)],
                        acc.at[pl.ds(s * RPT, RPT)])
        plsc.subcore_barrier()

        def group(gi, carry):
            descs = []
            for b in range(K):
                d = pltpu.async_copy(ones, acc.at[slab.at[gi * K + b]],
                                     sem, add=True)
                descs.append(d)
            for d in descs:
                d.wait()
            return carry
        lax.fori_loop(0, nch // K, group, 0)

        plsc.subcore_barrier()
        pltpu.sync_copy(acc.at[pl.ds(s * RPT, RPT)],
                        deg_out.at[pl.ds(c * NP + s * RPT, RPT)])
    return body


def _edge_kernel_body(nch):
    ngroups = nch // K
    assert ngroups % 2 == 0

    def body(idx_hbm, table_hbm, zeros_hbm, s_out, ibuf, rows, acc,
             isem, gsem, ssem0, ssem1):
        c = lax.axis_index("c")
        s = lax.axis_index("s")
        wid = c * NT + s
        ssems = (ssem0, ssem1)
        pltpu.sync_copy(zeros_hbm.at[pl.ds(s * RPT, RPT)],
                        acc.at[pl.ds(s * RPT, RPT)])
        plsc.subcore_barrier()

        pltpu.async_copy(idx_hbm.at[wid, pl.ds(0, 2 * K)], ibuf.at[0], isem)

        def drain_bank(slot, ib):
            for k in range(K):
                pltpu.make_async_copy(rows.at[slot, k],
                                      acc.at[ibuf.at[ib, 2 * k + 1]],
                                      ssems[slot]).wait()

        def outer(g2, carry):
            for slot in range(2):
                g = g2 * 2 + slot
                ib = lax.rem(g, 4)
                pltpu.make_async_copy(idx_hbm.at[wid, pl.ds(g * 2 * K, 2 * K)],
                                      ibuf.at[ib], isem).wait()

                @pl.when(g + 1 < ngroups)
                def _():
                    pltpu.async_copy(
                        idx_hbm.at[wid, pl.ds((g + 1) * 2 * K, 2 * K)],
                        ibuf.at[lax.rem(g + 1, 4)], isem)

                @pl.when(g2 >= 1)
                def _():
                    drain_bank(slot, ib)

                gdescs = []
                for k in range(K):
                    d = pltpu.async_copy(table_hbm.at[ibuf.at[ib, 2 * k]],
                                         rows.at[slot, k], gsem)
                    gdescs.append(d)
                for k in range(K):
                    gdescs[k].wait()
                    pltpu.async_copy(rows.at[slot, k],
                                     acc.at[ibuf.at[ib, 2 * k + 1]],
                                     ssems[slot], add=True)
            return carry
        lax.fori_loop(0, ngroups // 2, outer, 0)
        for slot in range(2):
            drain_bank(slot, slot)

        plsc.subcore_barrier()
        pltpu.sync_copy(acc.at[pl.ds(s * RPT, RPT)],
                        s_out.at[pl.ds(c * NP + s * RPT, RPT)])
    return body


def _pack(v):
    r = v.shape[0]
    y = v.reshape(r // PN, PN, TW)
    return jnp.concatenate([y[:, a, :] for a in range(PN)], axis=1)


def _scale_tc_kernel(xu_ref, xi_ref, w_ref, deg_ref, g_ref, d_ref):
    for c, xref in enumerate((xu_ref, xi_ref)):
        h = lax.dot_general(xref[...], w_ref[c], (((0,), (0,)), ((), ())),
                            preferred_element_type=jnp.float32)
        dinv = lax.rsqrt(deg_ref[c] + 1.0)
        g_ref[c] = _pack(h * dinv[:, None])
        d_ref[c] = _pack(dinv[:, None] + jnp.zeros_like(h))


def _decode_tc_kernel(g_ref, s_ref, d_ref, b_ref, o_ref):
    grp = lax.broadcasted_iota(jnp.int32, (128, PN), 0)
    gcol = lax.broadcasted_iota(jnp.int32, (128, PN), 1)
    ssum = (grp // TW == gcol).astype(jnp.float32)

    def branch(c):
        bt = jnp.concatenate([b_ref[c]] * PN)
        return jnp.tanh(d_ref[c] * (s_ref[c] + g_ref[c]) + bt[None, :])
    prod = branch(0) * branch(1)
    rs = lax.dot(prod, ssum,
                 precision=lax.Precision.HIGHEST)
    z = rs.reshape(rs.shape[0] // TW, TW, PN)
    o_ref[...] = jnp.concatenate([z[:, a, :] for a in range(TW)], axis=1)


def kernel(x_user, adj_user, x_item, adj_item, W_user, b_user, W_item, b_item):
    n, d_in = x_user.shape
    e = adj_user.shape[1]
    nch = -(-e // (NT * CH))
    nch = -(-nch // (2 * K)) * (2 * K)
    ept = nch * CH
    pad = NT * ept - e

    def prep(adj, offset):
        src = adj[0].astype(jnp.int32) + offset
        dst = adj[1].astype(jnp.int32)
        src = jnp.concatenate(
            [src, jnp.full((pad,), offset + n, jnp.int32)])
        dst = jnp.concatenate([dst, jnp.full((pad,), n, jnp.int32)])
        return src.reshape(NT, nch, CH), dst.reshape(NT, nch, CH)

    su, du = prep(adj_user, 0)
    si, di = prep(adj_item, NP)
    src_all = jnp.concatenate([su, si], axis=0)
    dst_all = jnp.concatenate([du, di], axis=0)
    idx_all = jnp.stack([src_all, dst_all], axis=2).reshape(
        NC * NT, 2 * nch, CH)

    zeros1 = jnp.zeros((NP,), jnp.float32)
    zeros2 = jnp.zeros((NP, TW), jnp.float32)

    deg_kernel = pl.kernel(
        _deg_kernel_body(nch),
        out_type=jax.ShapeDtypeStruct((NC * NP,), jnp.float32),
        mesh=_mesh(),
        scratch_types=[
            pltpu.VMEM((nch, CH), jnp.int32),
            pltpu.VMEM((CH,), jnp.float32),
            pltpu.VMEM_SHARED((NP,), jnp.float32),
            pltpu.SemaphoreType.DMA,
        ],
        compiler_params=pltpu.CompilerParams(use_tc_tiling_on_sc=False),
    )
    deg = deg_kernel(dst_all, zeros1)

    colpad = jnp.zeros((d_in, NP - n), jnp.float32)
    xu_t = jnp.concatenate([x_user.T, colpad], axis=1)
    xi_t = jnp.concatenate([x_item.T, colpad], axis=1)
    w_all = jnp.stack([W_user, W_item])
    deg2 = deg.reshape(NC, NP)

    BR = 2048
    nb = NP // BR
    pr = NP * TW // 128
    bpr = BR * TW // 128
    g_all, d_all = pl.pallas_call(
        _scale_tc_kernel,
        grid=(nb,),
        in_specs=[
            pl.BlockSpec((d_in, BR), lambda i: (0, i)),
            pl.BlockSpec((d_in, BR), lambda i: (0, i)),
            pl.BlockSpec((NC, d_in, 8), lambda i: (0, 0, 0)),
            pl.BlockSpec((NC, BR), lambda i: (0, i)),
        ],
        out_specs=[
            pl.BlockSpec((NC, bpr, 128), lambda i: (0, i, 0)),
            pl.BlockSpec((NC, bpr, 128), lambda i: (0, i, 0)),
        ],
        out_shape=[
            jax.ShapeDtypeStruct((NC, pr, 128), jnp.float32),
            jax.ShapeDtypeStruct((NC, pr, 128), jnp.float32),
        ],
    )(xu_t, xi_t, w_all, deg2)

    table = g_all.reshape(NC * NP, TW)

    edge_kernel = pl.kernel(
        _edge_kernel_body(nch),
        out_type=jax.ShapeDtypeStruct((NC * NP, TW), jnp.float32),
        mesh=_mesh(),
        scratch_types=[
            pltpu.VMEM((4, 2 * K, CH), jnp.int32),
            pltpu.VMEM((2, K, CH, TW), jnp.float32),
            pltpu.VMEM_SHARED((NP, TW), jnp.float32),
            pltpu.SemaphoreType.DMA,
            pltpu.SemaphoreType.DMA,
            pltpu.SemaphoreType.DMA,
            pltpu.SemaphoreType.DMA,
        ],
        compiler_params=pltpu.CompilerParams(use_tc_tiling_on_sc=False),
    )
    s_acc = edge_kernel(idx_all, table, zeros2)

    b_all = jnp.stack([b_user, b_item])
    score = pl.pallas_call(
        _decode_tc_kernel,
        grid=(nb,),
        in_specs=[
            pl.BlockSpec((NC, bpr, 128), lambda i: (0, i, 0)),
            pl.BlockSpec((NC, bpr, 128), lambda i: (0, i, 0)),
            pl.BlockSpec((NC, bpr, 128), lambda i: (0, i, 0)),
            pl.BlockSpec((NC, 8), lambda i: (0, 0)),
        ],
        out_specs=pl.BlockSpec((BR // 128, 128), lambda i: (i, 0)),
        out_shape=jax.ShapeDtypeStruct((NP // 128, 128), jnp.float32),
    )(g_all, s_acc.reshape(NC, pr, 128), d_all, b_all)

    return score.reshape(NP)[:n]
